# Initial kernel scaffold; baseline (speedup 1.0000x reference)
import jax
import jax.numpy as jnp
from jax.experimental import pallas as pl


def kernel(node_attr, edge_index, edge_attr, node_to_graph, select_reactant, num_reactant_batch, num_product_batch, W_proj, b_proj, W_bond, b_bond, gnn_bias, W_ih, W_hh, b_ih, b_hh, W_sp, b_sp, prelu_a):
    raise NotImplementedError("write your pallas kernel here")


# speedup vs baseline: 3.4789x; 3.4789x over previous
import functools

import numpy as np
import jax
import jax.numpy as jnp
from jax import lax
from jax.experimental import pallas as pl
from jax.experimental.pallas import tpu as pltpu
from jax.experimental.pallas import tpu_sc as plsc

F32 = jnp.float32


def _proj(x, w, b):
    n, d = x.shape
    h = w.shape[1]
    blk = 1000

    def body(x_ref, w_ref, b_ref, o_ref):
        o_ref[...] = jnp.maximum(
            jnp.dot(x_ref[...], w_ref[...], preferred_element_type=F32)
            + b_ref[...], 0.0)

    return pl.pallas_call(
        body,
        grid=(n // blk,),
        in_specs=[
            pl.BlockSpec((blk, d), lambda i: (i, 0)),
            pl.BlockSpec((d, h), lambda i: (0, 0)),
            pl.BlockSpec((1, h), lambda i: (0, 0)),
        ],
        out_specs=pl.BlockSpec((blk, h), lambda i: (i, 0)),
        out_shape=jax.ShapeDtypeStruct((n, h), F32),
    )(x, w, b)


def _msg(h_src, ea_aug, w_msg, t_rep, s_mat):
    e, h = h_src.shape
    k = ea_aug.shape[1]
    kh = k * h
    blk = 2000

    def body(hs_ref, ea_ref, wm_ref, tr_ref, s_ref, o_ref):
        p = jnp.dot(hs_ref[...], wm_ref[...], preferred_element_type=F32)
        r = jnp.dot(ea_ref[...], tr_ref[...], preferred_element_type=F32)
        o_ref[...] = jnp.dot(p * r, s_ref[...], preferred_element_type=F32)

    return pl.pallas_call(
        body,
        grid=(e // blk,),
        in_specs=[
            pl.BlockSpec((blk, h), lambda i: (i, 0)),
            pl.BlockSpec((blk, k), lambda i: (i, 0)),
            pl.BlockSpec((h, kh), lambda i: (0, 0)),
            pl.BlockSpec((k, kh), lambda i: (0, 0)),
            pl.BlockSpec((kh, h), lambda i: (0, 0)),
        ],
        out_specs=pl.BlockSpec((blk, h), lambda i: (i, 0)),
        out_shape=jax.ShapeDtypeStruct((e, h), F32),
    )(h_src, ea_aug, w_msg, t_rep, s_mat)


def _gru(aggp, hid, w_iht, w_hht, b_ih, b_hh, gbias):
    n, h = hid.shape
    blk = 1000

    def body(a_ref, h_ref, wi_ref, wh_ref, bi_ref, bh_ref, gb_ref, o_ref):
        x = jnp.maximum(a_ref[0] + a_ref[1] + gb_ref[...], 0.0)
        gi = jnp.dot(x, wi_ref[...], preferred_element_type=F32) + bi_ref[...]
        gh = jnp.dot(h_ref[...], wh_ref[...], preferred_element_type=F32) + bh_ref[...]
        r = jax.nn.sigmoid(gi[:, :h] + gh[:, :h])
        z = jax.nn.sigmoid(gi[:, h:2 * h] + gh[:, h:2 * h])
        nn = jnp.tanh(gi[:, 2 * h:] + r * gh[:, 2 * h:])
        o_ref[...] = (1.0 - z) * nn + z * h_ref[...]

    return pl.pallas_call(
        body,
        grid=(n // blk,),
        in_specs=[
            pl.BlockSpec((2, blk, h), lambda i: (0, i, 0)),
            pl.BlockSpec((blk, h), lambda i: (i, 0)),
            pl.BlockSpec((h, 3 * h), lambda i: (0, 0)),
            pl.BlockSpec((h, 3 * h), lambda i: (0, 0)),
            pl.BlockSpec((1, 3 * h), lambda i: (0, 0)),
            pl.BlockSpec((1, 3 * h), lambda i: (0, 0)),
            pl.BlockSpec((1, h), lambda i: (0, 0)),
        ],
        out_specs=pl.BlockSpec((blk, h), lambda i: (i, 0)),
        out_shape=jax.ShapeDtypeStruct((n, h), F32),
    )(aggp, hid, w_iht, w_hht, b_ih, b_hh, gbias)


def _pool(h, h0, ids3, w_sp_h, w_sp_h0, b_sp, m_r, m_p, a_prelu):
    n, hh = h.shape
    blk = 1000
    ngrid = n // blk
    g = m_r.shape[1]
    b = m_r.shape[0]
    d = w_sp_h.shape[1]

    def body(h_ref, h0_ref, id_ref, wh_ref, wh0_ref, bs_ref, mr_ref, mp_ref,
             a_ref, o_ref, mh_ref, mh0_ref):
        i = pl.program_id(0)

        @pl.when(i == 0)
        def _init():
            mh_ref[...] = jnp.zeros_like(mh_ref)
            mh0_ref[...] = jnp.zeros_like(mh0_ref)

        ids = id_ref[0]
        gi = lax.broadcasted_iota(jnp.int32, (g, blk), 0)
        oh = (gi == ids).astype(F32)
        mh_ref[...] += jnp.dot(oh, h_ref[...], preferred_element_type=F32)
        mh0_ref[...] += jnp.dot(oh, h0_ref[...], preferred_element_type=F32)

        @pl.when(i == ngrid - 1)
        def _fin():
            rx = (jnp.dot(mh_ref[...], wh_ref[...], preferred_element_type=F32)
                  + jnp.dot(mh0_ref[...], wh0_ref[...], preferred_element_type=F32)
                  + bs_ref[...])
            rx = jnp.where(rx > 0, rx, a_ref[0, 0] * rx)
            o_ref[...] = jnp.concatenate(
                [jnp.dot(mr_ref[...], rx, preferred_element_type=F32),
                 jnp.dot(mp_ref[...], rx, preferred_element_type=F32)], axis=1)

    return pl.pallas_call(
        body,
        grid=(ngrid,),
        in_specs=[
            pl.BlockSpec((blk, hh), lambda i: (i, 0)),
            pl.BlockSpec((blk, hh), lambda i: (i, 0)),
            pl.BlockSpec((1, 1, blk), lambda i: (i, 0, 0)),
            pl.BlockSpec((hh, d), lambda i: (0, 0)),
            pl.BlockSpec((hh, d), lambda i: (0, 0)),
            pl.BlockSpec((1, d), lambda i: (0, 0)),
            pl.BlockSpec((b, g), lambda i: (0, 0)),
            pl.BlockSpec((b, g), lambda i: (0, 0)),
            pl.BlockSpec((1, 1), lambda i: (0, 0)),
        ],
        out_specs=pl.BlockSpec((b, 2 * d), lambda i: (0, 0)),
        out_shape=jax.ShapeDtypeStruct((b, 2 * d), F32),
        scratch_shapes=[pltpu.VMEM((g, hh), F32), pltpu.VMEM((g, hh), F32)],
    )(h, h0, ids3, w_sp_h, w_sp_h0, b_sp, m_r, m_p, a_prelu)


_NW = 32
_NC = 2
_NS = 16
_CH = 125
_GRP = 20


def _sc_gather(table, src2):
    n, h = table.shape
    nchunk = src2.shape[0]
    t_per = nchunk // _NW
    assert t_per % _GRP == 0
    mesh = plsc.VectorSubcoreMesh(core_axis_name="c", subcore_axis_name="s")

    @functools.partial(
        pl.kernel,
        out_type=jax.ShapeDtypeStruct((nchunk, _CH, h), F32),
        mesh=mesh,
        compiler_params=pltpu.CompilerParams(use_tc_tiling_on_sc=False),
        scratch_types=[
            pltpu.VMEM((t_per, _CH), jnp.int32),
            pltpu.VMEM((_GRP, _CH, h), F32),
            pltpu.SemaphoreType.DMA,
        ],
    )
    def gather(table_hbm, src_hbm, out_hbm, idx_v, rows_v, sem):
        c = lax.axis_index("c")
        s = lax.axis_index("s")
        wid = s * _NC + c
        start = wid * t_per
        pltpu.sync_copy(src_hbm.at[pl.ds(start, t_per)], idx_v)
        for grp in range(t_per // _GRP):
            descs = []
            for j in range(_GRP):
                descs.append(pltpu.async_copy(
                    table_hbm.at[idx_v.at[grp * _GRP + j]], rows_v.at[j], sem))
            for dsc in descs:
                dsc.wait()
            pltpu.sync_copy(rows_v, out_hbm.at[pl.ds(start + grp * _GRP, _GRP)])

    return gather(table, src2)


def _sc_scatter(msg3, dst2, zeros_nh):
    nchunk = msg3.shape[0]
    h = msg3.shape[2]
    n = zeros_nh.shape[0]
    t_per = nchunk // _NW
    assert t_per % _GRP == 0
    rows_per_sub = n // _NS
    mesh = plsc.VectorSubcoreMesh(core_axis_name="c", subcore_axis_name="s")

    @functools.partial(
        pl.kernel,
        out_type=jax.ShapeDtypeStruct((_NC * n, h), F32),
        mesh=mesh,
        compiler_params=pltpu.CompilerParams(use_tc_tiling_on_sc=False),
        scratch_types=[
            pltpu.VMEM((t_per, _CH), jnp.int32),
            pltpu.VMEM((_GRP, _CH, h), F32),
            pltpu.VMEM_SHARED((n, h), F32),
            pltpu.SemaphoreType.DMA,
        ],
    )
    def scatter(msg_hbm, dst_hbm, zero_hbm, out_hbm, idx_v, msg_v, acc_sh, sem):
        c = lax.axis_index("c")
        s = lax.axis_index("s")
        row0 = s * rows_per_sub
        pltpu.sync_copy(zero_hbm.at[pl.ds(row0, rows_per_sub)],
                        acc_sh.at[pl.ds(row0, rows_per_sub)])
        plsc.subcore_barrier()
        start = (c * _NS + s) * t_per
        pltpu.sync_copy(dst_hbm.at[pl.ds(start, t_per)], idx_v)
        for grp in range(t_per // _GRP):
            pltpu.sync_copy(msg_hbm.at[pl.ds(start + grp * _GRP, _GRP)], msg_v)
            descs = []
            for j in range(_GRP):
                descs.append(pltpu.async_copy(
                    msg_v.at[j], acc_sh.at[idx_v.at[grp * _GRP + j]], sem,
                    add=True))
            for dsc in descs:
                dsc.wait()
        plsc.subcore_barrier()
        pltpu.sync_copy(acc_sh.at[pl.ds(row0, rows_per_sub)],
                        out_hbm.at[pl.ds(c * n + row0, rows_per_sub)])

    return scatter(msg3, dst2, zeros_nh)


def kernel(node_attr, edge_index, edge_attr, node_to_graph, select_reactant,
           num_reactant_batch, num_product_batch,
           W_proj, b_proj, W_bond, b_bond, gnn_bias,
           W_ih, W_hh, b_ih, b_hh, W_sp, b_sp, prelu_a):
    n, d_node = node_attr.shape
    e = edge_index.shape[1]
    d_edge = edge_attr.shape[1]
    h = W_proj.shape[1]
    k = d_edge + 1
    b = num_reactant_batch.shape[0]
    g = select_reactant.shape[0]
    d_hid = W_sp.shape[1]

    nchunk = e // _CH
    src2 = edge_index[0].reshape(nchunk, _CH).astype(jnp.int32)
    dst2 = edge_index[1].reshape(nchunk, _CH).astype(jnp.int32)
    ea_aug = jnp.concatenate([edge_attr, jnp.ones((e, 1), F32)], axis=1)

    wb3 = jnp.concatenate(
        [W_bond.reshape(d_edge, h, h), b_bond.reshape(1, h, h)], axis=0)
    w_msg = wb3.transpose(1, 0, 2).reshape(h, k * h)
    t_rep = jnp.asarray(np.kron(np.eye(k, dtype=np.float32),
                                np.ones((1, h), np.float32)))
    s_mat = jnp.asarray(np.tile(np.eye(h, dtype=np.float32), (k, 1)))

    h0 = _proj(node_attr, W_proj, b_proj.reshape(1, h))

    w_iht = W_ih.T
    w_hht = W_hh.T
    zeros_nh = jnp.zeros((n, h), F32)

    hid = h0
    for _ in range(3):
        hsrc3 = _sc_gather(hid, src2)
        msg = _msg(hsrc3.reshape(e, h), ea_aug, w_msg, t_rep, s_mat)
        aggp = _sc_scatter(msg.reshape(nchunk, _CH, h), dst2, zeros_nh)
        hid = _gru(aggp.reshape(2, n, h), hid, w_iht, w_hht,
                   b_ih.reshape(1, 3 * h), b_hh.reshape(1, 3 * h),
                   gnn_bias.reshape(1, h))

    r_idx = jnp.nonzero(select_reactant, size=b)[0]
    p_idx = jnp.nonzero(jnp.logical_not(select_reactant), size=b)[0]
    seg_r = jnp.repeat(jnp.arange(b), num_reactant_batch, total_repeat_length=b)
    seg_p = jnp.repeat(jnp.arange(b), num_product_batch, total_repeat_length=b)
    m_r = jnp.zeros((b, g), F32).at[seg_r, r_idx].add(1.0)
    m_p = jnp.zeros((b, g), F32).at[seg_p, p_idx].add(1.0)

    ids3 = node_to_graph.astype(jnp.int32).reshape(n // 1000, 1, 1000)
    out = _pool(hid, h0, ids3, W_sp[:h], W_sp[h:], b_sp.reshape(1, d_hid),
                m_r, m_p, jnp.reshape(prelu_a, (1, 1)))
    return out


# speedup vs baseline: 5.2090x; 1.4973x over previous
import functools

import numpy as np
import jax
import jax.numpy as jnp
from jax import lax
from jax.experimental import pallas as pl
from jax.experimental.pallas import tpu as pltpu
from jax.experimental.pallas import tpu_sc as plsc

F32 = jnp.float32


def _proj(x, w, b):
    n, d = x.shape
    h = w.shape[1]
    blk = 1000

    def body(x_ref, w_ref, b_ref, o_ref):
        o_ref[...] = jnp.maximum(
            jnp.dot(x_ref[...], w_ref[...], preferred_element_type=F32)
            + b_ref[...], 0.0)

    return pl.pallas_call(
        body,
        grid=(n // blk,),
        in_specs=[
            pl.BlockSpec((blk, d), lambda i: (i, 0)),
            pl.BlockSpec((d, h), lambda i: (0, 0)),
            pl.BlockSpec((1, h), lambda i: (0, 0)),
        ],
        out_specs=pl.BlockSpec((blk, h), lambda i: (i, 0)),
        out_shape=jax.ShapeDtypeStruct((n, h), F32),
    )(x, w, b)


def _msg(hs4, ea4, w_big, t_rep4, s_big, b_big):
    e4 = hs4.shape[0]
    kw = w_big.shape[1]
    ke = ea4.shape[1]
    blk = 800

    def body(hs_ref, ea_ref, wb_ref, tr_ref, sb_ref, bb_ref, o_ref):
        p = jnp.dot(hs_ref[...], wb_ref[...], preferred_element_type=F32)
        r = jnp.dot(ea_ref[...], tr_ref[...], preferred_element_type=F32)
        o_ref[...] = (jnp.dot(p * r, sb_ref[...], preferred_element_type=F32)
                      + jnp.dot(hs_ref[...], bb_ref[...],
                                preferred_element_type=F32))

    return pl.pallas_call(
        body,
        grid=(e4 // blk,),
        in_specs=[
            pl.BlockSpec((blk, 128), lambda i: (i, 0)),
            pl.BlockSpec((blk, ke), lambda i: (i, 0)),
            pl.BlockSpec((128, kw), lambda i: (0, 0)),
            pl.BlockSpec((ke, kw), lambda i: (0, 0)),
            pl.BlockSpec((kw, 128), lambda i: (0, 0)),
            pl.BlockSpec((128, 128), lambda i: (0, 0)),
        ],
        out_specs=pl.BlockSpec((blk, 128), lambda i: (i, 0)),
        out_shape=jax.ShapeDtypeStruct((e4, 128), F32),
    )(hs4, ea4, w_big, t_rep4, s_big, b_big)


def _gru(aggp, hid, w_iht, w_hht, b_ih, b_hh, gbias):
    n, h = hid.shape
    blk = 1000

    def body(a_ref, h_ref, wi_ref, wh_ref, bi_ref, bh_ref, gb_ref, o_ref):
        x = jnp.maximum(a_ref[0] + a_ref[1] + gb_ref[...], 0.0)
        gi = jnp.dot(x, wi_ref[...], preferred_element_type=F32) + bi_ref[...]
        gh = jnp.dot(h_ref[...], wh_ref[...], preferred_element_type=F32) + bh_ref[...]
        r = jax.nn.sigmoid(gi[:, :h] + gh[:, :h])
        z = jax.nn.sigmoid(gi[:, h:2 * h] + gh[:, h:2 * h])
        nn = jnp.tanh(gi[:, 2 * h:] + r * gh[:, 2 * h:])
        o_ref[...] = (1.0 - z) * nn + z * h_ref[...]

    return pl.pallas_call(
        body,
        grid=(n // blk,),
        in_specs=[
            pl.BlockSpec((2, blk, h), lambda i: (0, i, 0)),
            pl.BlockSpec((blk, h), lambda i: (i, 0)),
            pl.BlockSpec((h, 3 * h), lambda i: (0, 0)),
            pl.BlockSpec((h, 3 * h), lambda i: (0, 0)),
            pl.BlockSpec((1, 3 * h), lambda i: (0, 0)),
            pl.BlockSpec((1, 3 * h), lambda i: (0, 0)),
            pl.BlockSpec((1, h), lambda i: (0, 0)),
        ],
        out_specs=pl.BlockSpec((blk, h), lambda i: (i, 0)),
        out_shape=jax.ShapeDtypeStruct((n, h), F32),
    )(aggp, hid, w_iht, w_hht, b_ih, b_hh, gbias)


def _pool(h, h0, ids3, w_sp_h, w_sp_h0, b_sp, m_r, m_p, a_prelu):
    n, hh = h.shape
    blk = 1000
    ngrid = n // blk
    g = m_r.shape[1]
    b = m_r.shape[0]
    d = w_sp_h.shape[1]

    def body(h_ref, h0_ref, id_ref, wh_ref, wh0_ref, bs_ref, mr_ref, mp_ref,
             a_ref, o_ref, mh_ref, mh0_ref):
        i = pl.program_id(0)

        @pl.when(i == 0)
        def _init():
            mh_ref[...] = jnp.zeros_like(mh_ref)
            mh0_ref[...] = jnp.zeros_like(mh0_ref)

        ids = id_ref[0]
        gi = lax.broadcasted_iota(jnp.int32, (g, blk), 0)
        oh = (gi == ids).astype(F32)
        mh_ref[...] += jnp.dot(oh, h_ref[...], preferred_element_type=F32)
        mh0_ref[...] += jnp.dot(oh, h0_ref[...], preferred_element_type=F32)

        @pl.when(i == ngrid - 1)
        def _fin():
            rx = (jnp.dot(mh_ref[...], wh_ref[...], preferred_element_type=F32)
                  + jnp.dot(mh0_ref[...], wh0_ref[...], preferred_element_type=F32)
                  + bs_ref[...])
            rx = jnp.where(rx > 0, rx, a_ref[0, 0] * rx)
            o_ref[...] = jnp.concatenate(
                [jnp.dot(mr_ref[...], rx, preferred_element_type=F32),
                 jnp.dot(mp_ref[...], rx, preferred_element_type=F32)], axis=1)

    return pl.pallas_call(
        body,
        grid=(ngrid,),
        in_specs=[
            pl.BlockSpec((blk, hh), lambda i: (i, 0)),
            pl.BlockSpec((blk, hh), lambda i: (i, 0)),
            pl.BlockSpec((1, 1, blk), lambda i: (i, 0, 0)),
            pl.BlockSpec((hh, d), lambda i: (0, 0)),
            pl.BlockSpec((hh, d), lambda i: (0, 0)),
            pl.BlockSpec((1, d), lambda i: (0, 0)),
            pl.BlockSpec((b, g), lambda i: (0, 0)),
            pl.BlockSpec((b, g), lambda i: (0, 0)),
            pl.BlockSpec((1, 1), lambda i: (0, 0)),
        ],
        out_specs=pl.BlockSpec((b, 2 * d), lambda i: (0, 0)),
        out_shape=jax.ShapeDtypeStruct((b, 2 * d), F32),
        scratch_shapes=[pltpu.VMEM((g, hh), F32), pltpu.VMEM((g, hh), F32)],
    )(h, h0, ids3, w_sp_h, w_sp_h0, b_sp, m_r, m_p, a_prelu)


_NW = 32
_NC = 2
_NS = 16
_CH = 125
_GRP = 20


def _sc_gather(table, src2):
    n, h = table.shape
    nchunk = src2.shape[0]
    t_per = nchunk // _NW
    assert t_per % _GRP == 0
    mesh = plsc.VectorSubcoreMesh(core_axis_name="c", subcore_axis_name="s")

    @functools.partial(
        pl.kernel,
        out_type=jax.ShapeDtypeStruct((nchunk, _CH, h), F32),
        mesh=mesh,
        compiler_params=pltpu.CompilerParams(use_tc_tiling_on_sc=False),
        scratch_types=[
            pltpu.VMEM((t_per, _CH), jnp.int32),
            pltpu.VMEM((_GRP, _CH, h), F32),
            pltpu.SemaphoreType.DMA,
        ],
    )
    def gather(table_hbm, src_hbm, out_hbm, idx_v, rows_v, sem):
        c = lax.axis_index("c")
        s = lax.axis_index("s")
        wid = s * _NC + c
        start = wid * t_per
        pltpu.sync_copy(src_hbm.at[pl.ds(start, t_per)], idx_v)
        for grp in range(t_per // _GRP):
            descs = []
            for j in range(_GRP):
                descs.append(pltpu.async_copy(
                    table_hbm.at[idx_v.at[grp * _GRP + j]], rows_v.at[j], sem))
            for dsc in descs:
                dsc.wait()
            pltpu.sync_copy(rows_v, out_hbm.at[pl.ds(start + grp * _GRP, _GRP)])

    return gather(table, src2)


def _sc_scatter(msg3, dst2, zeros_nh):
    nchunk = msg3.shape[0]
    h = msg3.shape[2]
    n = zeros_nh.shape[0]
    t_per = nchunk // _NW
    assert t_per % _GRP == 0
    rows_per_sub = n // _NS
    mesh = plsc.VectorSubcoreMesh(core_axis_name="c", subcore_axis_name="s")

    @functools.partial(
        pl.kernel,
        out_type=jax.ShapeDtypeStruct((_NC * n, h), F32),
        mesh=mesh,
        compiler_params=pltpu.CompilerParams(use_tc_tiling_on_sc=False),
        scratch_types=[
            pltpu.VMEM((t_per, _CH), jnp.int32),
            pltpu.VMEM((_GRP, _CH, h), F32),
            pltpu.VMEM_SHARED((n, h), F32),
            pltpu.SemaphoreType.DMA,
        ],
    )
    def scatter(msg_hbm, dst_hbm, zero_hbm, out_hbm, idx_v, msg_v, acc_sh, sem):
        c = lax.axis_index("c")
        s = lax.axis_index("s")
        row0 = s * rows_per_sub
        pltpu.sync_copy(zero_hbm.at[pl.ds(row0, rows_per_sub)],
                        acc_sh.at[pl.ds(row0, rows_per_sub)])
        plsc.subcore_barrier()
        start = (c * _NS + s) * t_per
        pltpu.sync_copy(dst_hbm.at[pl.ds(start, t_per)], idx_v)
        for grp in range(t_per // _GRP):
            pltpu.sync_copy(msg_hbm.at[pl.ds(start + grp * _GRP, _GRP)], msg_v)
            descs = []
            for j in range(_GRP):
                descs.append(pltpu.async_copy(
                    msg_v.at[j], acc_sh.at[idx_v.at[grp * _GRP + j]], sem,
                    add=True))
            for dsc in descs:
                dsc.wait()
        plsc.subcore_barrier()
        pltpu.sync_copy(acc_sh.at[pl.ds(row0, rows_per_sub)],
                        out_hbm.at[pl.ds(c * n + row0, rows_per_sub)])

    return scatter(msg3, dst2, zeros_nh)


def kernel(node_attr, edge_index, edge_attr, node_to_graph, select_reactant,
           num_reactant_batch, num_product_batch,
           W_proj, b_proj, W_bond, b_bond, gnn_bias,
           W_ih, W_hh, b_ih, b_hh, W_sp, b_sp, prelu_a):
    n, d_node = node_attr.shape
    e = edge_index.shape[1]
    d_edge = edge_attr.shape[1]
    h = W_proj.shape[1]
    k = d_edge + 1
    b = num_reactant_batch.shape[0]
    g = select_reactant.shape[0]
    d_hid = W_sp.shape[1]

    nchunk = e // _CH
    src2 = edge_index[0].reshape(nchunk, _CH).astype(jnp.int32)
    dst2 = edge_index[1].reshape(nchunk, _CH).astype(jnp.int32)
    ea4 = edge_attr.reshape(e // 4, 4 * d_edge)

    w_msg = W_bond.reshape(d_edge, h, h).transpose(1, 0, 2).reshape(h, d_edge * h)
    w_big = jax.scipy.linalg.block_diag(w_msg, w_msg, w_msg, w_msg)
    t_rep = np.kron(np.eye(d_edge, dtype=np.float32), np.ones((1, h), np.float32))
    t_rep4 = jnp.asarray(np.kron(np.eye(4, dtype=np.float32), t_rep))
    s_mat = np.tile(np.eye(h, dtype=np.float32), (d_edge, 1))
    s_big = jnp.asarray(np.kron(np.eye(4, dtype=np.float32), s_mat))
    bb = b_bond.reshape(h, h)
    b_big = jax.scipy.linalg.block_diag(bb, bb, bb, bb)

    h0 = _proj(node_attr, W_proj, b_proj.reshape(1, h))

    w_iht = W_ih.T
    w_hht = W_hh.T
    zeros_nh = jnp.zeros((n, h), F32)

    hid = h0
    for _ in range(3):
        hsrc3 = _sc_gather(hid, src2)
        msg4 = _msg(hsrc3.reshape(e // 4, 4 * h), ea4, w_big, t_rep4,
                    s_big, b_big)
        aggp = _sc_scatter(msg4.reshape(nchunk, _CH, h), dst2, zeros_nh)
        hid = _gru(aggp.reshape(2, n, h), hid, w_iht, w_hht,
                   b_ih.reshape(1, 3 * h), b_hh.reshape(1, 3 * h),
                   gnn_bias.reshape(1, h))

    r_idx = jnp.nonzero(select_reactant, size=b)[0]
    p_idx = jnp.nonzero(jnp.logical_not(select_reactant), size=b)[0]
    seg_r = jnp.repeat(jnp.arange(b), num_reactant_batch, total_repeat_length=b)
    seg_p = jnp.repeat(jnp.arange(b), num_product_batch, total_repeat_length=b)
    ar = jnp.arange(b)[None, :]
    ag = jnp.arange(g)[None, :]
    m_r = jnp.dot((seg_r[:, None] == ar).astype(F32).T,
                  (r_idx[:, None] == ag).astype(F32))
    m_p = jnp.dot((seg_p[:, None] == ar).astype(F32).T,
                  (p_idx[:, None] == ag).astype(F32))

    ids3 = node_to_graph.astype(jnp.int32).reshape(n // 1000, 1, 1000)
    out = _pool(hid, h0, ids3, W_sp[:h], W_sp[h:], b_sp.reshape(1, d_hid),
                m_r, m_p, jnp.reshape(prelu_a, (1, 1)))
    return out


# speedup vs baseline: 6.8930x; 1.3233x over previous
import functools

import numpy as np
import jax
import jax.numpy as jnp
from jax import lax
from jax.experimental import pallas as pl
from jax.experimental.pallas import tpu as pltpu
from jax.experimental.pallas import tpu_sc as plsc

F32 = jnp.float32


def _proj(x, w, b):
    n, d = x.shape
    h = w.shape[1]
    blk = 1000

    def body(x_ref, w_ref, b_ref, o_ref):
        o_ref[...] = jnp.maximum(
            jnp.dot(x_ref[...], w_ref[...], preferred_element_type=F32)
            + b_ref[...], 0.0)

    return pl.pallas_call(
        body,
        grid=(n // blk,),
        in_specs=[
            pl.BlockSpec((blk, d), lambda i: (i, 0)),
            pl.BlockSpec((d, h), lambda i: (0, 0)),
            pl.BlockSpec((1, h), lambda i: (0, 0)),
        ],
        out_specs=pl.BlockSpec((blk, h), lambda i: (i, 0)),
        out_shape=jax.ShapeDtypeStruct((n, h), F32),
    )(x, w, b)


def _msg(hs4, ea4, wk_stack, ek_stack, b_big):
    e4 = hs4.shape[0]
    dk = wk_stack.shape[0]
    ke = ea4.shape[1]
    blk = 800

    def body(hs_ref, ea_ref, wk_ref, ek_ref, bb_ref, o_ref):
        hs = hs_ref[...]
        ea = ea_ref[...]
        acc = jnp.dot(hs, bb_ref[...], preferred_element_type=F32)
        for kk in range(dk):
            p = jnp.dot(hs, wk_ref[kk], preferred_element_type=F32)
            r = jnp.dot(ea, ek_ref[kk], preferred_element_type=F32)
            acc += p * r
        o_ref[...] = acc

    return pl.pallas_call(
        body,
        grid=(e4 // blk,),
        in_specs=[
            pl.BlockSpec((blk, 128), lambda i: (i, 0)),
            pl.BlockSpec((blk, ke), lambda i: (i, 0)),
            pl.BlockSpec((dk, 128, 128), lambda i: (0, 0, 0)),
            pl.BlockSpec((dk, ke, 128), lambda i: (0, 0, 0)),
            pl.BlockSpec((128, 128), lambda i: (0, 0)),
        ],
        out_specs=pl.BlockSpec((blk, 128), lambda i: (i, 0)),
        out_shape=jax.ShapeDtypeStruct((e4, 128), F32),
    )(hs4, ea4, wk_stack, ek_stack, b_big)


def _gru(aggp, hid, w_iht, w_hht, b_ih, b_hh, gbias):
    n, h = hid.shape
    blk = 1000

    def body(a_ref, h_ref, wi_ref, wh_ref, bi_ref, bh_ref, gb_ref, o_ref):
        x = jnp.maximum(a_ref[0] + a_ref[1] + gb_ref[...], 0.0)
        gi = jnp.dot(x, wi_ref[...], preferred_element_type=F32) + bi_ref[...]
        gh = jnp.dot(h_ref[...], wh_ref[...], preferred_element_type=F32) + bh_ref[...]
        r = jax.nn.sigmoid(gi[:, :h] + gh[:, :h])
        z = jax.nn.sigmoid(gi[:, h:2 * h] + gh[:, h:2 * h])
        nn = jnp.tanh(gi[:, 2 * h:] + r * gh[:, 2 * h:])
        o_ref[...] = (1.0 - z) * nn + z * h_ref[...]

    return pl.pallas_call(
        body,
        grid=(n // blk,),
        in_specs=[
            pl.BlockSpec((2, blk, h), lambda i: (0, i, 0)),
            pl.BlockSpec((blk, h), lambda i: (i, 0)),
            pl.BlockSpec((h, 3 * h), lambda i: (0, 0)),
            pl.BlockSpec((h, 3 * h), lambda i: (0, 0)),
            pl.BlockSpec((1, 3 * h), lambda i: (0, 0)),
            pl.BlockSpec((1, 3 * h), lambda i: (0, 0)),
            pl.BlockSpec((1, h), lambda i: (0, 0)),
        ],
        out_specs=pl.BlockSpec((blk, h), lambda i: (i, 0)),
        out_shape=jax.ShapeDtypeStruct((n, h), F32),
    )(aggp, hid, w_iht, w_hht, b_ih, b_hh, gbias)


def _pool(h, h0, ids3, w_sp_h, w_sp_h0, b_sp, m_r, m_p, a_prelu):
    n, hh = h.shape
    blk = 1000
    ngrid = n // blk
    g = m_r.shape[1]
    b = m_r.shape[0]
    d = w_sp_h.shape[1]

    def body(h_ref, h0_ref, id_ref, wh_ref, wh0_ref, bs_ref, mr_ref, mp_ref,
             a_ref, o_ref, mh_ref, mh0_ref):
        i = pl.program_id(0)

        @pl.when(i == 0)
        def _init():
            mh_ref[...] = jnp.zeros_like(mh_ref)
            mh0_ref[...] = jnp.zeros_like(mh0_ref)

        ids = id_ref[0]
        gi = lax.broadcasted_iota(jnp.int32, (g, blk), 0)
        oh = (gi == ids).astype(F32)
        mh_ref[...] += jnp.dot(oh, h_ref[...], preferred_element_type=F32)
        mh0_ref[...] += jnp.dot(oh, h0_ref[...], preferred_element_type=F32)

        @pl.when(i == ngrid - 1)
        def _fin():
            rx = (jnp.dot(mh_ref[...], wh_ref[...], preferred_element_type=F32)
                  + jnp.dot(mh0_ref[...], wh0_ref[...], preferred_element_type=F32)
                  + bs_ref[...])
            rx = jnp.where(rx > 0, rx, a_ref[0, 0] * rx)
            o_ref[...] = jnp.concatenate(
                [jnp.dot(mr_ref[...], rx, preferred_element_type=F32),
                 jnp.dot(mp_ref[...], rx, preferred_element_type=F32)], axis=1)

    return pl.pallas_call(
        body,
        grid=(ngrid,),
        in_specs=[
            pl.BlockSpec((blk, hh), lambda i: (i, 0)),
            pl.BlockSpec((blk, hh), lambda i: (i, 0)),
            pl.BlockSpec((1, 1, blk), lambda i: (i, 0, 0)),
            pl.BlockSpec((hh, d), lambda i: (0, 0)),
            pl.BlockSpec((hh, d), lambda i: (0, 0)),
            pl.BlockSpec((1, d), lambda i: (0, 0)),
            pl.BlockSpec((b, g), lambda i: (0, 0)),
            pl.BlockSpec((b, g), lambda i: (0, 0)),
            pl.BlockSpec((1, 1), lambda i: (0, 0)),
        ],
        out_specs=pl.BlockSpec((b, 2 * d), lambda i: (0, 0)),
        out_shape=jax.ShapeDtypeStruct((b, 2 * d), F32),
        scratch_shapes=[pltpu.VMEM((g, hh), F32), pltpu.VMEM((g, hh), F32)],
    )(h, h0, ids3, w_sp_h, w_sp_h0, b_sp, m_r, m_p, a_prelu)


_NW = 32
_NC = 2
_NS = 16
_CH = 125
_GRP = 20


def _sc_gather(table, src2):
    n, h = table.shape
    nchunk = src2.shape[0]
    t_per = nchunk // _NW
    assert t_per % _GRP == 0
    mesh = plsc.VectorSubcoreMesh(core_axis_name="c", subcore_axis_name="s")

    @functools.partial(
        pl.kernel,
        out_type=jax.ShapeDtypeStruct((nchunk, _CH, h), F32),
        mesh=mesh,
        compiler_params=pltpu.CompilerParams(use_tc_tiling_on_sc=False),
        scratch_types=[
            pltpu.VMEM((t_per, _CH), jnp.int32),
            pltpu.VMEM((_GRP, _CH, h), F32),
            pltpu.SemaphoreType.DMA,
        ],
    )
    def gather(table_hbm, src_hbm, out_hbm, idx_v, rows_v, sem):
        c = lax.axis_index("c")
        s = lax.axis_index("s")
        wid = s * _NC + c
        start = wid * t_per
        pltpu.sync_copy(src_hbm.at[pl.ds(start, t_per)], idx_v)
        for grp in range(t_per // _GRP):
            descs = []
            for j in range(_GRP):
                descs.append(pltpu.async_copy(
                    table_hbm.at[idx_v.at[grp * _GRP + j]], rows_v.at[j], sem))
            for dsc in descs:
                dsc.wait()
            pltpu.sync_copy(rows_v, out_hbm.at[pl.ds(start + grp * _GRP, _GRP)])

    return gather(table, src2)


def _sc_scatter(msg3, dst2, zeros_nh):
    nchunk = msg3.shape[0]
    h = msg3.shape[2]
    n = zeros_nh.shape[0]
    t_per = nchunk // _NW
    assert t_per % _GRP == 0
    rows_per_sub = n // _NS
    mesh = plsc.VectorSubcoreMesh(core_axis_name="c", subcore_axis_name="s")

    @functools.partial(
        pl.kernel,
        out_type=jax.ShapeDtypeStruct((_NC * n, h), F32),
        mesh=mesh,
        compiler_params=pltpu.CompilerParams(use_tc_tiling_on_sc=False),
        scratch_types=[
            pltpu.VMEM((t_per, _CH), jnp.int32),
            pltpu.VMEM((_GRP, _CH, h), F32),
            pltpu.VMEM_SHARED((n, h), F32),
            pltpu.SemaphoreType.DMA,
        ],
    )
    def scatter(msg_hbm, dst_hbm, zero_hbm, out_hbm, idx_v, msg_v, acc_sh, sem):
        c = lax.axis_index("c")
        s = lax.axis_index("s")
        row0 = s * rows_per_sub
        pltpu.sync_copy(zero_hbm.at[pl.ds(row0, rows_per_sub)],
                        acc_sh.at[pl.ds(row0, rows_per_sub)])
        plsc.subcore_barrier()
        start = (c * _NS + s) * t_per
        pltpu.sync_copy(dst_hbm.at[pl.ds(start, t_per)], idx_v)
        for grp in range(t_per // _GRP):
            pltpu.sync_copy(msg_hbm.at[pl.ds(start + grp * _GRP, _GRP)], msg_v)
            descs = []
            for j in range(_GRP):
                descs.append(pltpu.async_copy(
                    msg_v.at[j], acc_sh.at[idx_v.at[grp * _GRP + j]], sem,
                    add=True))
            for dsc in descs:
                dsc.wait()
        plsc.subcore_barrier()
        pltpu.sync_copy(acc_sh.at[pl.ds(row0, rows_per_sub)],
                        out_hbm.at[pl.ds(c * n + row0, rows_per_sub)])

    return scatter(msg3, dst2, zeros_nh)


def kernel(node_attr, edge_index, edge_attr, node_to_graph, select_reactant,
           num_reactant_batch, num_product_batch,
           W_proj, b_proj, W_bond, b_bond, gnn_bias,
           W_ih, W_hh, b_ih, b_hh, W_sp, b_sp, prelu_a):
    n, d_node = node_attr.shape
    e = edge_index.shape[1]
    d_edge = edge_attr.shape[1]
    h = W_proj.shape[1]
    k = d_edge + 1
    b = num_reactant_batch.shape[0]
    g = select_reactant.shape[0]
    d_hid = W_sp.shape[1]

    nchunk = e // _CH
    src2 = edge_index[0].reshape(nchunk, _CH).astype(jnp.int32)
    dst2 = edge_index[1].reshape(nchunk, _CH).astype(jnp.int32)
    ea4 = edge_attr.reshape(e // 4, 4 * d_edge)

    eye4 = jnp.eye(4, dtype=F32)
    wb = W_bond.reshape(d_edge, h, h)
    wk_stack = jax.vmap(lambda m: jnp.kron(eye4, m))(wb)
    ek_stack = jnp.asarray(np.stack([
        np.kron(np.eye(4, dtype=np.float32),
                np.eye(d_edge, dtype=np.float32)[:, [kk]]
                @ np.ones((1, h), np.float32))
        for kk in range(d_edge)]))
    bb = b_bond.reshape(h, h)
    b_big = jax.scipy.linalg.block_diag(bb, bb, bb, bb)

    h0 = _proj(node_attr, W_proj, b_proj.reshape(1, h))

    w_iht = W_ih.T
    w_hht = W_hh.T
    zeros_nh = jnp.zeros((n, h), F32)

    hid = h0
    for _ in range(3):
        hsrc3 = _sc_gather(hid, src2)
        msg4 = _msg(hsrc3.reshape(e // 4, 4 * h), ea4, wk_stack, ek_stack,
                    b_big)
        aggp = _sc_scatter(msg4.reshape(nchunk, _CH, h), dst2, zeros_nh)
        hid = _gru(aggp.reshape(2, n, h), hid, w_iht, w_hht,
                   b_ih.reshape(1, 3 * h), b_hh.reshape(1, 3 * h),
                   gnn_bias.reshape(1, h))

    r_idx = jnp.nonzero(select_reactant, size=b)[0]
    p_idx = jnp.nonzero(jnp.logical_not(select_reactant), size=b)[0]
    seg_r = jnp.repeat(jnp.arange(b), num_reactant_batch, total_repeat_length=b)
    seg_p = jnp.repeat(jnp.arange(b), num_product_batch, total_repeat_length=b)
    ar = jnp.arange(b)[None, :]
    ag = jnp.arange(g)[None, :]
    m_r = jnp.dot((seg_r[:, None] == ar).astype(F32).T,
                  (r_idx[:, None] == ag).astype(F32))
    m_p = jnp.dot((seg_p[:, None] == ar).astype(F32).T,
                  (p_idx[:, None] == ag).astype(F32))

    ids3 = node_to_graph.astype(jnp.int32).reshape(n // 1000, 1, 1000)
    out = _pool(hid, h0, ids3, W_sp[:h], W_sp[h:], b_sp.reshape(1, d_hid),
                m_r, m_p, jnp.reshape(prelu_a, (1, 1)))
    return out
